# Initial kernel scaffold; baseline (speedup 1.0000x reference)
import jax
import jax.numpy as jnp
from jax.experimental import pallas as pl


def kernel(pooled_energy, node_ids, n_unpooled):
    raise NotImplementedError("write your pallas kernel here")


# speedup vs baseline: 1.2308x; 1.2308x over previous
import jax
import jax.numpy as jnp
from jax import lax
from jax.experimental import pallas as pl
from jax.experimental.pallas import tpu as pltpu
from jax.experimental.pallas import tpu_sc as plsc

N_POOLED = 50000
N_UNPOOLED = 100000
D = 512
NC, NS, L = 2, 16, 16
NW = NC * NS
RANGE = N_UNPOOLED // NW
IDS_CHUNK = 2000
N_ID_CHUNKS = N_POOLED // IDS_CHUNK
VECS_PER_CHUNK = IDS_CHUNK // L
ZROWS = 64
N_ZFULL = RANGE // ZROWS
ZTAIL = RANGE - N_ZFULL * ZROWS
ZWAVE = 8
CAP = 3200
CHUNK = 16


def _splat(x, idx):
    return lax.gather(
        x, idx[:, None],
        dimension_numbers=lax.GatherDimensionNumbers(
            offset_dims=(), collapsed_slice_dims=(0,), start_index_map=(0,)),
        slice_sizes=(1,),
        mode=lax.GatherScatterMode.PROMISE_IN_BOUNDS)


def _body(pooled_hbm, ids_hbm, zeros_hbm, pad_hbm, out_hbm,
          idsbuf, srcbuf, dstbuf, padbuf, zbuf, rowbuf,
          sem_z, sem_g, sem_s):
    cid = lax.axis_index("c")
    sid = lax.axis_index("s")
    wid = sid * NC + cid
    base = wid * RANGE

    pltpu.sync_copy(zeros_hbm, zbuf)
    pltpu.sync_copy(pad_hbm, padbuf)

    def zero_wave(wv, _):
        cps = [
            pltpu.async_copy(
                zbuf, out_hbm.at[pl.ds(base + (wv * ZWAVE + j) * ZROWS, ZROWS)],
                sem_z)
            for j in range(ZWAVE)
        ]
        for cp in cps:
            cp.wait()
        return 0
    lax.fori_loop(0, N_ZFULL // ZWAVE, zero_wave, 0)
    pltpu.sync_copy(zbuf.at[pl.ds(0, ZTAIL)],
                    out_hbm.at[pl.ds(base + N_ZFULL * ZROWS, ZTAIL)])

    lane15 = jnp.full((L,), 15, jnp.int32)

    def id_chunk(cc, nv_vec):
        pltpu.sync_copy(ids_hbm.at[pl.ds(cc * IDS_CHUNK, IDS_CHUNK)], idsbuf)

        def vec_body(i, nv_vec):
            v = idsbuf[pl.ds(i * L, L)]
            m = (v >= base) & (v < base + RANGE)
            pf = plsc.cumsum(m.astype(jnp.int32))
            pos = nv_vec + pf - 1
            plsc.store_scatter(dstbuf, [pos], v, mask=m)
            srcs = lax.iota(jnp.int32, L) + (cc * IDS_CHUNK + i * L)
            plsc.store_scatter(srcbuf, [pos], srcs, mask=m)
            cnt = _splat(pf, lane15)
            return nv_vec + cnt

        return lax.fori_loop(0, VECS_PER_CHUNK, vec_body, nv_vec)

    nv_vec = lax.fori_loop(0, N_ID_CHUNKS, id_chunk, jnp.zeros((L,), jnp.int32))
    nv = jnp.max(nv_vec)

    padpos = nv + lax.iota(jnp.int32, L)
    plsc.store_scatter(dstbuf, [padpos], padbuf[...])
    plsc.store_scatter(srcbuf, [padpos], jnp.zeros((L,), jnp.int32))

    trips = (nv + CHUNK - 1) // CHUNK

    def sc_body(t, _):
        sv = srcbuf[pl.ds(t * CHUNK, CHUNK)]
        dv = dstbuf[pl.ds(t * CHUNK, CHUNK)]
        pltpu.async_copy(pooled_hbm.at[sv], rowbuf, sem_g).wait()
        pltpu.async_copy(rowbuf, out_hbm.at[dv], sem_s).wait()
        return 0

    lax.fori_loop(0, trips, sc_body, 0)


def kernel(pooled_energy, node_ids, n_unpooled):
    ids32 = node_ids.astype(jnp.int32)
    zeros_in = jnp.zeros((ZROWS, D), jnp.float32)
    pad_in = jnp.broadcast_to(ids32[0], (L,))
    call = pl.kernel(
        _body,
        out_type=jax.ShapeDtypeStruct((N_UNPOOLED, D), jnp.float32),
        mesh=plsc.VectorSubcoreMesh(core_axis_name="c", subcore_axis_name="s"),
        compiler_params=pltpu.CompilerParams(
            use_tc_tiling_on_sc=False, needs_layout_passes=False),
        scratch_types=[
            pltpu.VMEM((IDS_CHUNK,), jnp.int32),
            pltpu.VMEM((CAP,), jnp.int32),
            pltpu.VMEM((CAP,), jnp.int32),
            pltpu.VMEM((L,), jnp.int32),
            pltpu.VMEM((ZROWS, D), jnp.float32),
            pltpu.VMEM((CHUNK, D), jnp.float32),
            pltpu.SemaphoreType.DMA,
            pltpu.SemaphoreType.DMA,
            pltpu.SemaphoreType.DMA,
        ],
    )
    return call(pooled_energy, ids32, zeros_in, pad_in)


# speedup vs baseline: 1.2709x; 1.0326x over previous
import jax
import jax.numpy as jnp
from jax import lax
from jax.experimental import pallas as pl
from jax.experimental.pallas import tpu as pltpu
from jax.experimental.pallas import tpu_sc as plsc

N_POOLED = 50000
N_UNPOOLED = 100000
D = 512
NC, NS, L = 2, 16, 16
NW = NC * NS
RANGE = N_UNPOOLED // NW
IDS_CHUNK = 2000
N_ID_CHUNKS = N_POOLED // IDS_CHUNK
VECS_PER_CHUNK = IDS_CHUNK // L
ZROWS = 64
N_ZFULL = RANGE // ZROWS
ZTAIL = RANGE - N_ZFULL * ZROWS
ZWAVE = 8
CAP = 3200
CHUNK = 64


def _splat(x, idx):
    return lax.gather(
        x, idx[:, None],
        dimension_numbers=lax.GatherDimensionNumbers(
            offset_dims=(), collapsed_slice_dims=(0,), start_index_map=(0,)),
        slice_sizes=(1,),
        mode=lax.GatherScatterMode.PROMISE_IN_BOUNDS)


def _body(pooled_hbm, ids_hbm, zeros_hbm, pad_hbm, out_hbm,
          idsbuf, srcbuf, dstbuf, padbuf, zbuf, rowA, rowB,
          sem_z, sem_g, sem_s):
    cid = lax.axis_index("c")
    sid = lax.axis_index("s")
    wid = sid * NC + cid
    base = wid * RANGE

    pltpu.sync_copy(zeros_hbm, zbuf)
    pltpu.sync_copy(pad_hbm, padbuf)

    def zero_wave(wv, _):
        cps = [
            pltpu.async_copy(
                zbuf, out_hbm.at[pl.ds(base + (wv * ZWAVE + j) * ZROWS, ZROWS)],
                sem_z)
            for j in range(ZWAVE)
        ]
        for cp in cps:
            cp.wait()
        return 0
    lax.fori_loop(0, N_ZFULL // ZWAVE, zero_wave, 0)
    pltpu.sync_copy(zbuf.at[pl.ds(0, ZTAIL)],
                    out_hbm.at[pl.ds(base + N_ZFULL * ZROWS, ZTAIL)])

    lane15 = jnp.full((L,), 15, jnp.int32)

    def id_chunk(cc, nv_vec):
        pltpu.sync_copy(ids_hbm.at[pl.ds(cc * IDS_CHUNK, IDS_CHUNK)], idsbuf)

        def vec_body(i, nv_vec):
            v = idsbuf[pl.ds(i * L, L)]
            m = (v >= base) & (v < base + RANGE)
            pf = plsc.cumsum(m.astype(jnp.int32))
            pos = nv_vec + pf - 1
            plsc.store_scatter(dstbuf, [pos], v, mask=m)
            srcs = lax.iota(jnp.int32, L) + (cc * IDS_CHUNK + i * L)
            plsc.store_scatter(srcbuf, [pos], srcs, mask=m)
            cnt = _splat(pf, lane15)
            return nv_vec + cnt

        return lax.fori_loop(0, VECS_PER_CHUNK, vec_body, nv_vec)

    nv_vec = lax.fori_loop(0, N_ID_CHUNKS, id_chunk, jnp.zeros((L,), jnp.int32))
    nv = jnp.max(nv_vec)

    for j in range(CHUNK // L):
        padpos = nv + lax.iota(jnp.int32, L) + j * L
        plsc.store_scatter(dstbuf, [padpos], padbuf[...])
        plsc.store_scatter(srcbuf, [padpos], jnp.zeros((L,), jnp.int32))

    trips = (nv + CHUNK - 1) // CHUNK

    def issue_gather(t, buf):
        pltpu.async_copy(
            pooled_hbm.at[srcbuf.at[pl.ds(t * CHUNK, CHUNK)]], buf, sem_g)

    def step(t, mine, other):
        pltpu.make_async_copy(
            pooled_hbm.at[srcbuf.at[pl.ds(0, CHUNK)]], mine, sem_g).wait()
        pltpu.async_copy(
            mine, out_hbm.at[dstbuf.at[pl.ds(t * CHUNK, CHUNK)]], sem_s)

        @pl.when(t >= 1)
        def _():
            pltpu.make_async_copy(
                other, out_hbm.at[dstbuf.at[pl.ds(0, CHUNK)]], sem_s).wait()

        @pl.when(t + 1 < trips)
        def _():
            issue_gather(t + 1, other)

    @pl.when(trips > 0)
    def _():
        issue_gather(0, rowA)

    def pipe(t, _):
        @pl.when(lax.rem(t, 2) == 0)
        def _():
            step(t, rowA, rowB)

        @pl.when(lax.rem(t, 2) == 1)
        def _():
            step(t, rowB, rowA)
        return 0

    lax.fori_loop(0, trips, pipe, 0)

    @pl.when(trips > 0)
    def _():
        pltpu.make_async_copy(
            rowA, out_hbm.at[dstbuf.at[pl.ds(0, CHUNK)]], sem_s).wait()


def kernel(pooled_energy, node_ids, n_unpooled):
    ids32 = node_ids.astype(jnp.int32)
    zeros_in = jnp.zeros((ZROWS, D), jnp.float32)
    pad_in = jnp.broadcast_to(ids32[0], (L,))
    call = pl.kernel(
        _body,
        out_type=jax.ShapeDtypeStruct((N_UNPOOLED, D), jnp.float32),
        mesh=plsc.VectorSubcoreMesh(core_axis_name="c", subcore_axis_name="s"),
        compiler_params=pltpu.CompilerParams(
            use_tc_tiling_on_sc=False, needs_layout_passes=False),
        scratch_types=[
            pltpu.VMEM((IDS_CHUNK,), jnp.int32),
            pltpu.VMEM((CAP,), jnp.int32),
            pltpu.VMEM((CAP,), jnp.int32),
            pltpu.VMEM((L,), jnp.int32),
            pltpu.VMEM((ZROWS, D), jnp.float32),
            pltpu.VMEM((CHUNK, D), jnp.float32),
            pltpu.VMEM((CHUNK, D), jnp.float32),
            pltpu.SemaphoreType.DMA,
            pltpu.SemaphoreType.DMA,
            pltpu.SemaphoreType.DMA,
        ],
    )
    return call(pooled_energy, ids32, zeros_in, pad_in)


# speedup vs baseline: 2.5058x; 1.9716x over previous
import jax
import jax.numpy as jnp
from jax import lax
from jax.experimental import pallas as pl
from jax.experimental.pallas import tpu as pltpu
from jax.experimental.pallas import tpu_sc as plsc

N_POOLED = 50000
N_UNPOOLED = 100000
D = 512
NC, NS, L = 2, 16, 16
NW = NC * NS
RANGE = N_UNPOOLED // NW
IDS_CHUNK = 2000
N_ID_CHUNKS = N_POOLED // IDS_CHUNK
VECS_PER_CHUNK = IDS_CHUNK // L
ZROWS = 64
N_ZFULL = RANGE // ZROWS
ZTAIL = RANGE - N_ZFULL * ZROWS
ZWAVE = 8
CAP = 3200
CHUNK = 64


def _splat(x, idx):
    return lax.gather(
        x, idx[:, None],
        dimension_numbers=lax.GatherDimensionNumbers(
            offset_dims=(), collapsed_slice_dims=(0,), start_index_map=(0,)),
        slice_sizes=(1,),
        mode=lax.GatherScatterMode.PROMISE_IN_BOUNDS)


def _body(pooled_hbm, ids_hbm, zeros_hbm, pad_hbm, out_hbm,
          idsbuf, srcbuf, dstbuf, padbuf, zbuf, rowA, rowB,
          sem_z, sem_g, sem_s):
    cid = lax.axis_index("c")
    sid = lax.axis_index("s")
    wid = sid * NC + cid
    base = wid * 3120 + jnp.minimum(wid, 20) * 8
    rangew = jnp.where(wid < 20, 3128, 3120)

    pltpu.sync_copy(zeros_hbm, zbuf)
    pltpu.sync_copy(pad_hbm, padbuf)

    def zero_wave(wv, _):
        cps = [
            pltpu.async_copy(
                zbuf, out_hbm.at[pl.ds(base + (wv * ZWAVE + j) * ZROWS, ZROWS)],
                sem_z)
            for j in range(ZWAVE)
        ]
        for cp in cps:
            cp.wait()
        return 0
    lax.fori_loop(0, N_ZFULL // ZWAVE, zero_wave, 0)

    @pl.when(wid < 20)
    def _():
        pltpu.sync_copy(zbuf.at[pl.ds(0, 56)],
                        out_hbm.at[pl.ds(base + N_ZFULL * ZROWS, 56)])

    @pl.when(wid >= 20)
    def _():
        pltpu.sync_copy(zbuf.at[pl.ds(0, 48)],
                        out_hbm.at[pl.ds(base + N_ZFULL * ZROWS, 48)])

    lane15 = jnp.full((L,), 15, jnp.int32)

    def id_chunk(cc, nv_vec):
        pltpu.sync_copy(ids_hbm.at[pl.ds(cc * IDS_CHUNK, IDS_CHUNK)], idsbuf)

        def vec_body(i, nv_vec):
            v = idsbuf[pl.ds(i * L, L)]
            m = (v >= base) & (v < base + rangew)
            pf = plsc.cumsum(m.astype(jnp.int32))
            pos = nv_vec + pf - 1
            plsc.store_scatter(dstbuf, [pos], v, mask=m)
            srcs = lax.iota(jnp.int32, L) + (cc * IDS_CHUNK + i * L)
            plsc.store_scatter(srcbuf, [pos], srcs, mask=m)
            cnt = _splat(pf, lane15)
            return nv_vec + cnt

        return lax.fori_loop(0, VECS_PER_CHUNK, vec_body, nv_vec)

    nv_vec = lax.fori_loop(0, N_ID_CHUNKS, id_chunk, jnp.zeros((L,), jnp.int32))
    nv = jnp.max(nv_vec)

    for j in range(CHUNK // L):
        padpos = nv + lax.iota(jnp.int32, L) + j * L
        plsc.store_scatter(dstbuf, [padpos], padbuf[...])
        plsc.store_scatter(srcbuf, [padpos], jnp.zeros((L,), jnp.int32))

    trips = (nv + CHUNK - 1) // CHUNK

    def issue_gather(t, buf):
        pltpu.async_copy(
            pooled_hbm.at[srcbuf.at[pl.ds(t * CHUNK, CHUNK)]], buf, sem_g)

    def step(t, mine, other):
        pltpu.make_async_copy(
            pooled_hbm.at[srcbuf.at[pl.ds(0, CHUNK)]], mine, sem_g).wait()
        pltpu.async_copy(
            mine, out_hbm.at[dstbuf.at[pl.ds(t * CHUNK, CHUNK)]], sem_s)

        @pl.when(t >= 1)
        def _():
            pltpu.make_async_copy(
                other, out_hbm.at[dstbuf.at[pl.ds(0, CHUNK)]], sem_s).wait()

        @pl.when(t + 1 < trips)
        def _():
            issue_gather(t + 1, other)

    @pl.when(trips > 0)
    def _():
        issue_gather(0, rowA)

    def pipe(t, _):
        @pl.when(lax.rem(t, 2) == 0)
        def _():
            step(t, rowA, rowB)

        @pl.when(lax.rem(t, 2) == 1)
        def _():
            step(t, rowB, rowA)
        return 0

    lax.fori_loop(0, trips, pipe, 0)

    @pl.when(trips > 0)
    def _():
        pltpu.make_async_copy(
            rowA, out_hbm.at[dstbuf.at[pl.ds(0, CHUNK)]], sem_s).wait()


def kernel(pooled_energy, node_ids, n_unpooled):
    ids32 = node_ids.astype(jnp.int32)
    zeros_in = jnp.zeros((ZROWS, D), jnp.float32)
    pad_in = jnp.broadcast_to(ids32[0], (L,))
    call = pl.kernel(
        _body,
        out_type=jax.ShapeDtypeStruct((N_UNPOOLED, D), jnp.float32),
        mesh=plsc.VectorSubcoreMesh(core_axis_name="c", subcore_axis_name="s"),
        compiler_params=pltpu.CompilerParams(needs_layout_passes=False),
        scratch_types=[
            pltpu.VMEM((IDS_CHUNK,), jnp.int32),
            pltpu.VMEM((CAP,), jnp.int32),
            pltpu.VMEM((CAP,), jnp.int32),
            pltpu.VMEM((L,), jnp.int32),
            pltpu.VMEM((ZROWS, D), jnp.float32),
            pltpu.VMEM((CHUNK, D), jnp.float32),
            pltpu.VMEM((CHUNK, D), jnp.float32),
            pltpu.SemaphoreType.DMA,
            pltpu.SemaphoreType.DMA,
            pltpu.SemaphoreType.DMA,
        ],
    )
    return call(pooled_energy, ids32, zeros_in, pad_in)


# speedup vs baseline: 2.8717x; 1.1460x over previous
import jax
import jax.numpy as jnp
from jax import lax
from jax.experimental import pallas as pl
from jax.experimental.pallas import tpu as pltpu
from jax.experimental.pallas import tpu_sc as plsc

N_POOLED = 50000
N_UNPOOLED = 100000
D = 512
NC, NS, L = 2, 16, 16
NW = NC * NS
RANGE = N_UNPOOLED // NW
IDS_CHUNK = 2000
N_ID_CHUNKS = N_POOLED // IDS_CHUNK
VECS_PER_CHUNK = IDS_CHUNK // L
ZROWS = 64
N_ZFULL = RANGE // ZROWS
ZTAIL = RANGE - N_ZFULL * ZROWS
ZWAVE = 8
CAP = 3200
CHUNK = 32
NBUF = 4


def _splat(x, idx):
    return lax.gather(
        x, idx[:, None],
        dimension_numbers=lax.GatherDimensionNumbers(
            offset_dims=(), collapsed_slice_dims=(0,), start_index_map=(0,)),
        slice_sizes=(1,),
        mode=lax.GatherScatterMode.PROMISE_IN_BOUNDS)


def _body(pooled_hbm, ids_hbm, zeros_hbm, pad_hbm, out_hbm,
          idsbuf, srcbuf, dstbuf, padbuf, zbuf, rowA, rowB, rowC, rowD,
          sem_z, sem_g, sem_s):
    cid = lax.axis_index("c")
    sid = lax.axis_index("s")
    wid = sid * NC + cid
    base = wid * 3120 + jnp.minimum(wid, 20) * 8
    rangew = jnp.where(wid < 20, 3128, 3120)

    pltpu.sync_copy(zeros_hbm, zbuf)
    pltpu.sync_copy(pad_hbm, padbuf)

    def zero_wave(wv, _):
        cps = [
            pltpu.async_copy(
                zbuf, out_hbm.at[pl.ds(base + (wv * ZWAVE + j) * ZROWS, ZROWS)],
                sem_z)
            for j in range(ZWAVE)
        ]
        for cp in cps:
            cp.wait()
        return 0
    lax.fori_loop(0, N_ZFULL // ZWAVE, zero_wave, 0)

    @pl.when(wid < 20)
    def _():
        pltpu.sync_copy(zbuf.at[pl.ds(0, 56)],
                        out_hbm.at[pl.ds(base + N_ZFULL * ZROWS, 56)])

    @pl.when(wid >= 20)
    def _():
        pltpu.sync_copy(zbuf.at[pl.ds(0, 48)],
                        out_hbm.at[pl.ds(base + N_ZFULL * ZROWS, 48)])

    lane15 = jnp.full((L,), 15, jnp.int32)

    def id_chunk(cc, nv_vec):
        pltpu.sync_copy(ids_hbm.at[pl.ds(cc * IDS_CHUNK, IDS_CHUNK)], idsbuf)

        def vec_body(i, nv_vec):
            v = idsbuf[pl.ds(i * L, L)]
            m = (v >= base) & (v < base + rangew)
            pf = plsc.cumsum(m.astype(jnp.int32))
            pos = nv_vec + pf - 1
            plsc.store_scatter(dstbuf, [pos], v, mask=m)
            srcs = lax.iota(jnp.int32, L) + (cc * IDS_CHUNK + i * L)
            plsc.store_scatter(srcbuf, [pos], srcs, mask=m)
            cnt = _splat(pf, lane15)
            return nv_vec + cnt

        return lax.fori_loop(0, VECS_PER_CHUNK, vec_body, nv_vec)

    nv_vec = lax.fori_loop(0, N_ID_CHUNKS, id_chunk, jnp.zeros((L,), jnp.int32))
    nv = jnp.max(nv_vec)

    for j in range(CHUNK // L):
        padpos = nv + lax.iota(jnp.int32, L) + j * L
        plsc.store_scatter(dstbuf, [padpos], padbuf[...])
        plsc.store_scatter(srcbuf, [padpos], jnp.zeros((L,), jnp.int32))

    trips = (nv + CHUNK - 1) // CHUNK
    bufs = [rowA, rowB, rowC, rowD]

    def issue_gather(t, buf):
        pltpu.async_copy(
            pooled_hbm.at[srcbuf.at[pl.ds(t * CHUNK, CHUNK)]], buf, sem_g)

    def wait_gather(buf):
        pltpu.make_async_copy(
            pooled_hbm.at[srcbuf.at[pl.ds(0, CHUNK)]], buf, sem_g).wait()

    def wait_scatter():
        pltpu.make_async_copy(
            rowA, out_hbm.at[dstbuf.at[pl.ds(0, CHUNK)]], sem_s).wait()

    def step(t, mine, ahead2):
        wait_gather(mine)
        pltpu.async_copy(
            mine, out_hbm.at[dstbuf.at[pl.ds(t * CHUNK, CHUNK)]], sem_s)

        @pl.when(t + 2 < trips)
        def _():
            @pl.when(t >= 2)
            def _():
                wait_scatter()
            issue_gather(t + 2, ahead2)

    @pl.when(trips > 0)
    def _():
        issue_gather(0, rowA)

    @pl.when(trips > 1)
    def _():
        issue_gather(1, rowB)

    def pipe(t, _):
        for r in range(NBUF):
            @pl.when(lax.rem(t, NBUF) == r)
            def _(r=r):
                step(t, bufs[r], bufs[(r + 2) % NBUF])
        return 0

    lax.fori_loop(0, trips, pipe, 0)

    lax.fori_loop(0, jnp.minimum(trips, NBUF), lambda i, _: (wait_scatter(), 0)[1], 0)


def kernel(pooled_energy, node_ids, n_unpooled):
    ids32 = node_ids.astype(jnp.int32)
    zeros_in = jnp.zeros((ZROWS, D), jnp.float32)
    pad_in = jnp.broadcast_to(ids32[0], (L,))
    call = pl.kernel(
        _body,
        out_type=jax.ShapeDtypeStruct((N_UNPOOLED, D), jnp.float32),
        mesh=plsc.VectorSubcoreMesh(core_axis_name="c", subcore_axis_name="s"),
        compiler_params=pltpu.CompilerParams(needs_layout_passes=False),
        scratch_types=[
            pltpu.VMEM((IDS_CHUNK,), jnp.int32),
            pltpu.VMEM((CAP,), jnp.int32),
            pltpu.VMEM((CAP,), jnp.int32),
            pltpu.VMEM((L,), jnp.int32),
            pltpu.VMEM((ZROWS, D), jnp.float32),
            pltpu.VMEM((CHUNK, D), jnp.float32),
            pltpu.VMEM((CHUNK, D), jnp.float32),
            pltpu.VMEM((CHUNK, D), jnp.float32),
            pltpu.VMEM((CHUNK, D), jnp.float32),
            pltpu.SemaphoreType.DMA,
            pltpu.SemaphoreType.DMA,
            pltpu.SemaphoreType.DMA,
        ],
    )
    return call(pooled_energy, ids32, zeros_in, pad_in)


# speedup vs baseline: 3.0585x; 1.0651x over previous
import jax
import jax.numpy as jnp
from jax import lax
from jax.experimental import pallas as pl
from jax.experimental.pallas import tpu as pltpu
from jax.experimental.pallas import tpu_sc as plsc

N_POOLED = 50000
N_UNPOOLED = 100000
D = 512
NC, NS, L = 2, 16, 16
NW = NC * NS
IDS_CHUNK = 2000
N_ID_CHUNKS = N_POOLED // IDS_CHUNK
VECS_PER_CHUNK = IDS_CHUNK // L
FLAG_VECS = 196
CAP = 3200
CHUNK = 32
NBUF = 4


def _splat(x, idx):
    return lax.gather(
        x, idx[:, None],
        dimension_numbers=lax.GatherDimensionNumbers(
            offset_dims=(), collapsed_slice_dims=(0,), start_index_map=(0,)),
        slice_sizes=(1,),
        mode=lax.GatherScatterMode.PROMISE_IN_BOUNDS)


def _body(pooled_hbm, ids_hbm, zeros_hbm, pad_hbm, out_hbm,
          idsbuf, srcbuf, dstbuf, zdstbuf, flagbuf, padbuf,
          zbuf, rowA, rowB, rowC, rowD,
          sem_z, sem_g, sem_s):
    cid = lax.axis_index("c")
    sid = lax.axis_index("s")
    wid = sid * NC + cid
    base = wid * 3120 + jnp.minimum(wid, 20) * 8
    rangew = jnp.where(wid < 20, 3128, 3120)

    pltpu.sync_copy(zeros_hbm, zbuf)
    pltpu.sync_copy(pad_hbm, padbuf)

    zero16 = jnp.zeros((L,), jnp.int32)

    def clear_body(i, _):
        flagbuf[pl.ds(i * L, L)] = zero16
        return 0
    lax.fori_loop(0, FLAG_VECS, clear_body, 0)

    lane15 = jnp.full((L,), 15, jnp.int32)
    one16 = jnp.ones((L,), jnp.int32)

    def id_chunk(cc, nv_vec):
        pltpu.sync_copy(ids_hbm.at[pl.ds(cc * IDS_CHUNK, IDS_CHUNK)], idsbuf)

        def vec_body(i, nv_vec):
            v = idsbuf[pl.ds(i * L, L)]
            m = (v >= base) & (v < base + rangew)
            pf = plsc.cumsum(m.astype(jnp.int32))
            pos = nv_vec + pf - 1
            plsc.store_scatter(dstbuf, [pos], v, mask=m)
            srcs = lax.iota(jnp.int32, L) + (cc * IDS_CHUNK + i * L)
            plsc.store_scatter(srcbuf, [pos], srcs, mask=m)
            plsc.store_scatter(flagbuf, [v - base], one16, mask=m)
            return nv_vec + _splat(pf, lane15)

        return lax.fori_loop(0, VECS_PER_CHUNK, vec_body, nv_vec)

    nv_vec = lax.fori_loop(0, N_ID_CHUNKS, id_chunk, jnp.zeros((L,), jnp.int32))
    nv = jnp.max(nv_vec)

    for j in range(CHUNK // L):
        padpos = nv + lax.iota(jnp.int32, L) + j * L
        plsc.store_scatter(dstbuf, [padpos], padbuf[...])
        plsc.store_scatter(srcbuf, [padpos], zero16)

    def comp_body(k, carry):
        nz_vec, zmax_vec = carry
        rel = lax.iota(jnp.int32, L) + k * L
        f = flagbuf[pl.ds(k * L, L)]
        mz = (f == 0) & (rel < rangew)
        absrow = base + rel
        pfz = plsc.cumsum(mz.astype(jnp.int32))
        plsc.store_scatter(zdstbuf, [nz_vec + pfz - 1], absrow, mask=mz)
        vals = jnp.where(mz, absrow, -1)
        zmax_vec = jnp.maximum(zmax_vec, _splat(plsc.cummax(vals), lane15))
        return nz_vec + _splat(pfz, lane15), zmax_vec

    nz_vec, zmax_vec = lax.fori_loop(
        0, FLAG_VECS, comp_body,
        (jnp.zeros((L,), jnp.int32), jnp.full((L,), -1, jnp.int32)))
    nz = jnp.max(nz_vec)
    for j in range(CHUNK // L):
        zpadpos = nz + lax.iota(jnp.int32, L) + j * L
        plsc.store_scatter(zdstbuf, [zpadpos], zmax_vec)

    trips = (nv + CHUNK - 1) // CHUNK
    tripsz = (nz + CHUNK - 1) // CHUNK
    bufs = [rowA, rowB, rowC, rowD]

    def issue_gather(t, buf):
        pltpu.async_copy(
            pooled_hbm.at[srcbuf.at[pl.ds(t * CHUNK, CHUNK)]], buf, sem_g)

    def wait_gather(buf):
        pltpu.make_async_copy(
            pooled_hbm.at[srcbuf.at[pl.ds(0, CHUNK)]], buf, sem_g).wait()

    def wait_scatter(sem):
        pltpu.make_async_copy(
            rowA, out_hbm.at[dstbuf.at[pl.ds(0, CHUNK)]], sem).wait()

    def step(t, mine, ahead2):
        wait_gather(mine)
        pltpu.async_copy(
            mine, out_hbm.at[dstbuf.at[pl.ds(t * CHUNK, CHUNK)]], sem_s)

        @pl.when(t + 2 < trips)
        def _():
            @pl.when(t >= 2)
            def _():
                wait_scatter(sem_s)
            issue_gather(t + 2, ahead2)

    @pl.when(trips > 0)
    def _():
        issue_gather(0, rowA)

    @pl.when(trips > 1)
    def _():
        issue_gather(1, rowB)

    def pipe(t, _):
        @pl.when(t < tripsz)
        def _():
            pltpu.async_copy(
                zbuf, out_hbm.at[zdstbuf.at[pl.ds(t * CHUNK, CHUNK)]], sem_z)

        @pl.when(t < trips)
        def _():
            for r in range(NBUF):
                @pl.when(lax.rem(t, NBUF) == r)
                def _(r=r):
                    step(t, bufs[r], bufs[(r + 2) % NBUF])
        return 0

    lax.fori_loop(0, jnp.maximum(trips, tripsz), pipe, 0)

    lax.fori_loop(0, jnp.minimum(trips, NBUF),
                  lambda i, _: (wait_scatter(sem_s), 0)[1], 0)
    lax.fori_loop(0, tripsz,
                  lambda i, _: (wait_scatter(sem_z), 0)[1], 0)


def kernel(pooled_energy, node_ids, n_unpooled):
    ids32 = node_ids.astype(jnp.int32)
    zeros_in = jnp.zeros((CHUNK, D), jnp.float32)
    pad_in = jnp.broadcast_to(ids32[0], (L,))
    call = pl.kernel(
        _body,
        out_type=jax.ShapeDtypeStruct((N_UNPOOLED, D), jnp.float32),
        mesh=plsc.VectorSubcoreMesh(core_axis_name="c", subcore_axis_name="s"),
        compiler_params=pltpu.CompilerParams(needs_layout_passes=False),
        scratch_types=[
            pltpu.VMEM((IDS_CHUNK,), jnp.int32),
            pltpu.VMEM((CAP,), jnp.int32),
            pltpu.VMEM((CAP,), jnp.int32),
            pltpu.VMEM((CAP,), jnp.int32),
            pltpu.VMEM((FLAG_VECS * L,), jnp.int32),
            pltpu.VMEM((L,), jnp.int32),
            pltpu.VMEM((CHUNK, D), jnp.float32),
            pltpu.VMEM((CHUNK, D), jnp.float32),
            pltpu.VMEM((CHUNK, D), jnp.float32),
            pltpu.VMEM((CHUNK, D), jnp.float32),
            pltpu.VMEM((CHUNK, D), jnp.float32),
            pltpu.SemaphoreType.DMA,
            pltpu.SemaphoreType.DMA,
            pltpu.SemaphoreType.DMA,
        ],
    )
    return call(pooled_energy, ids32, zeros_in, pad_in)
